# baseline (speedup 1.0000x reference)
import functools

import jax
import jax.numpy as jnp
from jax import lax
from jax.experimental import pallas as pl
from jax.experimental.pallas import tpu as pltpu
from jax.experimental.pallas import tpu_sc as plsc

_N = 10000
_E = 320000
_C = 128
_K = 5
_NC, _NS = 2, 16
_NP = 10240
_NW = _NC * _NS
_EPW = _E // _NW
_CH128 = 128
_EP = _NW * 10240
_CPP = 10240 // _CH128
_RPT = _NP // _NS
_BR = 1280
_GRID = _NP // _BR


def _mesh():
    return plsc.VectorSubcoreMesh(core_axis_name="c", subcore_axis_name="s",
                                  num_cores=_NC, num_subcores=_NS)


_CHD = 80
_NCHD = _EPW // _CHD


def _deg_body(r_hbm, c_hbm, zf_hbm, mask_hbm, out_hbm,
              idx_v, m1_v, m2_v, acc):
    cid = lax.axis_index("c")
    sid = lax.axis_index("s")
    wid = cid * jnp.int32(_NS) + sid
    rows = pl.ds(sid * jnp.int32(_RPT), _RPT)
    pltpu.sync_copy(zf_hbm.at[rows], acc.at[rows])
    pltpu.sync_copy(mask_hbm.at[jnp.int32(0)], m1_v)
    pltpu.sync_copy(mask_hbm.at[jnp.int32(1)], m2_v)
    plsc.subcore_barrier()
    base = wid * jnp.int32(_EPW)

    def body(i, carry):
        off = base + i * jnp.int32(_CHD)
        pltpu.sync_copy(r_hbm.at[pl.ds(off, _CHD)], idx_v.at[jnp.int32(0)])
        pltpu.sync_copy(c_hbm.at[pl.ds(off, _CHD)], idx_v.at[jnp.int32(1)])
        pltpu.sync_copy(m1_v, acc.at[idx_v.at[jnp.int32(0)]], add=True)
        pltpu.sync_copy(m2_v, acc.at[idx_v.at[jnp.int32(1)]], add=True)
        return carry

    lax.fori_loop(jnp.int32(0), jnp.int32(_NCHD), body, jnp.int32(0))
    plsc.subcore_barrier()
    pltpu.sync_copy(acc.at[rows], out_hbm.at[cid, rows])


def _sc_degrees(r, c, zf, masks):
    f = pl.kernel(
        _deg_body,
        out_type=jax.ShapeDtypeStruct((_NC, _NP, _C), jnp.float32),
        mesh=_mesh(),
        scratch_types=[
            pltpu.VMEM((2, _CHD), jnp.int32),
            pltpu.VMEM((_CHD, _C), jnp.float32),
            pltpu.VMEM((_CHD, _C), jnp.float32),
            pltpu.VMEM_SHARED((_NP, _C), jnp.float32),
        ],
    )
    return f(r, c, zf, masks)


def _prop_body(u_hbm, gi_hbm, si_hbm, zf_hbm, out_hbm,
               gidx_v, sidx_v, buf0, acc, sem0):
    cid = lax.axis_index("c")
    sid = lax.axis_index("s")
    wid = cid * jnp.int32(_NS) + sid
    rows = pl.ds(sid * jnp.int32(_RPT), _RPT)
    pltpu.sync_copy(zf_hbm.at[rows], acc.at[rows])
    cbase = wid * jnp.int32(_CPP)
    pltpu.sync_copy(gi_hbm.at[pl.ds(cbase, _CPP)], gidx_v)
    pltpu.sync_copy(si_hbm.at[pl.ds(cbase, _CPP)], sidx_v)
    plsc.subcore_barrier()

    def body(j, carry):
        pltpu.async_copy(u_hbm.at[gidx_v.at[j]], buf0, sem0).wait()
        pltpu.sync_copy(buf0, acc.at[sidx_v.at[j]], add=True)
        return carry

    lax.fori_loop(jnp.int32(0), jnp.int32(_CPP), body, jnp.int32(0))
    plsc.subcore_barrier()
    pltpu.sync_copy(acc.at[rows], out_hbm.at[cid, rows])


def _sc_prop(u, gi, si, zf):
    f = pl.kernel(
        _prop_body,
        out_type=jax.ShapeDtypeStruct((_NC, _NP, _C), jnp.float32),
        mesh=_mesh(),
        scratch_types=[
            pltpu.VMEM((_CPP, _CH128), jnp.int32),
            pltpu.VMEM((_CPP, _CH128), jnp.int32),
            pltpu.VMEM((_CH128, _C), jnp.float32),
            pltpu.VMEM_SHARED((_NP, _C), jnp.float32),
            pltpu.SemaphoreType.DMA,
        ],
    )
    return f(u, gi, si, zf)


def _prep_body(deg_ref, x_ref, s1_ref, s2_ref, u1_ref, u2_ref):
    dsum = deg_ref[0] + deg_ref[1]
    d1 = dsum[:, 0:1]
    d2 = dsum[:, 64:65]
    s1 = jnp.where(d1 > 0.5, lax.rsqrt(jnp.maximum(d1, 1.0)), 0.0)
    s2 = jnp.where(d2 > 0.5, lax.rsqrt(jnp.maximum(d2, 1.0)), 0.0)
    s1b = jnp.broadcast_to(s1, (_BR, _C))
    s2b = jnp.broadcast_to(s2, (_BR, _C))
    s1_ref[...] = s1b
    s2_ref[...] = s2b
    u1_ref[...] = s1b * x_ref[...]
    u2_ref[...] = s2b * x_ref[...]


def _prep(degp, x):
    fb = jax.ShapeDtypeStruct((_NP, _C), jnp.float32)
    return pl.pallas_call(
        _prep_body,
        grid=(_GRID,),
        in_specs=[
            pl.BlockSpec((_NC, _BR, _C), lambda i: (jnp.int32(0), i, jnp.int32(0))),
            pl.BlockSpec((_BR, _C), lambda i: (i, jnp.int32(0))),
        ],
        out_specs=[pl.BlockSpec((_BR, _C), lambda i: (i, jnp.int32(0)))] * 4,
        out_shape=[fb, fb, fb, fb],
    )(degp, x)


def _combine_body(alpha, beta, zp_ref, s_ref, tm1_ref, tm2_ref, t_ref, u_ref):
    z = zp_ref[0] + zp_ref[1]
    s = s_ref[...]
    p = (-2.0 / 3.0) * s * z - (1.0 / 3.0) * tm1_ref[...]
    t = alpha * p - beta * tm2_ref[...]
    t_ref[...] = t
    u_ref[...] = s * t


def _combine(alpha, beta, zp, s, tm1, tm2):
    fb = jax.ShapeDtypeStruct((_NP, _C), jnp.float32)
    return pl.pallas_call(
        functools.partial(_combine_body, alpha, beta),
        grid=(_GRID,),
        in_specs=[
            pl.BlockSpec((_NC, _BR, _C), lambda i: (jnp.int32(0), i, jnp.int32(0))),
            pl.BlockSpec((_BR, _C), lambda i: (i, jnp.int32(0))),
            pl.BlockSpec((_BR, _C), lambda i: (i, jnp.int32(0))),
            pl.BlockSpec((_BR, _C), lambda i: (i, jnp.int32(0))),
        ],
        out_specs=[pl.BlockSpec((_BR, _C), lambda i: (i, jnp.int32(0)))] * 2,
        out_shape=[fb, fb],
    )(zp, s, tm1, tm2)


def _mm_body(w1_ref, w2_ref, b1_ref, b2_ref, *refs):
    t_refs, out_ref = refs[:-1], refs[-1]
    h = _C // 2
    acc1 = jnp.zeros((_BR, h), jnp.float32) + b1_ref[...]
    acc2 = jnp.zeros((_BR, h), jnp.float32) + b2_ref[...]
    for k in range(_K):
        acc1 = acc1 + jnp.dot(t_refs[k][...], w1_ref[k],
                              preferred_element_type=jnp.float32)
        acc2 = acc2 + jnp.dot(t_refs[_K + k][...], w2_ref[k],
                              preferred_element_type=jnp.float32)
    out_ref[...] = jnp.concatenate([acc1, acc2], axis=-1)


def _matmul(W1, W2, b1, b2, T1, T2):
    h = _C // 2
    tspec = pl.BlockSpec((_BR, _C), lambda i: (i, jnp.int32(0)))
    return pl.pallas_call(
        _mm_body,
        grid=(_GRID,),
        in_specs=[
            pl.BlockSpec((_K, _C, h), lambda i: (jnp.int32(0), jnp.int32(0), jnp.int32(0))),
            pl.BlockSpec((_K, _C, h), lambda i: (jnp.int32(0), jnp.int32(0), jnp.int32(0))),
            pl.BlockSpec((1, h), lambda i: (jnp.int32(0), jnp.int32(0))),
            pl.BlockSpec((1, h), lambda i: (jnp.int32(0), jnp.int32(0))),
        ] + [tspec] * (2 * _K),
        out_specs=pl.BlockSpec((_BR, _C), lambda i: (i, jnp.int32(0))),
        out_shape=jax.ShapeDtypeStruct((_NP, _C), jnp.float32),
    )(W1, W2, b1, b2, *T1, *T2)


def kernel(x, edge_index, W1, b1, W2, b2):
    x = jnp.zeros((_NP, _C), jnp.float32).at[:_N].set(x.astype(jnp.float32))
    ei = edge_index.astype(jnp.int32)
    r, c = ei[0], ei[1]
    zf = jnp.zeros((_NP, _C), jnp.float32)
    masks = jnp.zeros((2, _CHD, _C), jnp.float32)
    masks = masks.at[0, :, : _C // 2].set(1.0).at[1, :, _C // 2 :].set(1.0)

    npad = _EP - _E
    rp = jnp.concatenate([r, jnp.full((npad,), _NP - 1, jnp.int32)])
    cp = jnp.concatenate([c, jnp.full((npad,), _NP - 1, jnp.int32)])
    rp2 = rp.reshape(_EP // _CH128, _CH128)
    cp2 = cp.reshape(_EP // _CH128, _CH128)
    degp = _sc_degrees(r, c, zf, masks)
    s1, s2, u1, u2 = _prep(degp, x)

    stacks = []
    for s, u0, gi, si in ((s1, u1, cp2, rp2), (s2, u2, rp2, cp2)):
        T = [x]
        u = u0
        for k in range(1, _K):
            zp = _sc_prop(u, gi, si, zf)
            alpha, beta = (1.0, 0.0) if k == 1 else (2.0, 1.0)
            tm2 = T[k - 2] if k >= 2 else x
            t, u = _combine(alpha, beta, zp, s, T[k - 1], tm2)
            T.append(t)
        stacks.append(T)

    out = _matmul(W1.astype(jnp.float32), W2.astype(jnp.float32),
                  b1.reshape(1, -1).astype(jnp.float32),
                  b2.reshape(1, -1).astype(jnp.float32),
                  stacks[0], stacks[1])
    return out[:_N].astype(jnp.float64)


# speedup vs baseline: 2.3881x; 2.3881x over previous
import functools

import jax
import jax.numpy as jnp
from jax import lax
from jax.experimental import pallas as pl
from jax.experimental.pallas import tpu as pltpu
from jax.experimental.pallas import tpu_sc as plsc

_N = 10000
_E = 320000
_C = 128
_K = 5
_NC, _NS = 2, 16
_NP = 10240
_NW = _NC * _NS
_EPW = _E // _NW
_CH128 = 128
_EP = _NW * 10240
_CPP = 10240 // _CH128
_RPT = _NP // _NS
_BR = 1280
_GRID = _NP // _BR


def _mesh():
    return plsc.VectorSubcoreMesh(core_axis_name="c", subcore_axis_name="s",
                                  num_cores=_NC, num_subcores=_NS)


_CHD = 80
_NCHD = _EPW // _CHD


def _deg_body(r_hbm, c_hbm, zf_hbm, mask_hbm, out_hbm,
              idx_v, m1_v, m2_v, acc):
    cid = lax.axis_index("c")
    sid = lax.axis_index("s")
    wid = cid * jnp.int32(_NS) + sid
    rows = pl.ds(sid * jnp.int32(_RPT), _RPT)
    pltpu.sync_copy(zf_hbm.at[rows], acc.at[rows])
    pltpu.sync_copy(mask_hbm.at[jnp.int32(0)], m1_v)
    pltpu.sync_copy(mask_hbm.at[jnp.int32(1)], m2_v)
    plsc.subcore_barrier()
    base = wid * jnp.int32(_EPW)

    def body(i, carry):
        off = base + i * jnp.int32(_CHD)
        pltpu.sync_copy(r_hbm.at[pl.ds(off, _CHD)], idx_v.at[jnp.int32(0)])
        pltpu.sync_copy(c_hbm.at[pl.ds(off, _CHD)], idx_v.at[jnp.int32(1)])
        pltpu.sync_copy(m1_v, acc.at[idx_v.at[jnp.int32(0)]], add=True)
        pltpu.sync_copy(m2_v, acc.at[idx_v.at[jnp.int32(1)]], add=True)
        return carry

    lax.fori_loop(jnp.int32(0), jnp.int32(_NCHD), body, jnp.int32(0))
    plsc.subcore_barrier()
    pltpu.sync_copy(acc.at[rows], out_hbm.at[cid, rows])


def _sc_degrees(r, c, zf, masks):
    f = pl.kernel(
        _deg_body,
        out_type=jax.ShapeDtypeStruct((_NC, _NP, _C), jnp.float32),
        mesh=_mesh(),
        scratch_types=[
            pltpu.VMEM((2, _CHD), jnp.int32),
            pltpu.VMEM((_CHD, _C), jnp.float32),
            pltpu.VMEM((_CHD, _C), jnp.float32),
            pltpu.VMEM_SHARED((_NP, _C), jnp.float32),
        ],
    )
    return f(r, c, zf, masks)


_CHP = 80
_NCHP = _EPW // _CHP


def _prop_body(u_hbm, gi_hbm, si_hbm, zf_hbm, out_hbm,
               idx_v, buf0, buf1, acc, sem0, sem1):
    cid = lax.axis_index("c")
    sid = lax.axis_index("s")
    wid = cid * jnp.int32(_NS) + sid
    rows = pl.ds(sid * jnp.int32(_RPT), _RPT)
    pltpu.sync_copy(zf_hbm.at[rows], acc.at[rows])
    plsc.subcore_barrier()
    base = wid * jnp.int32(_EPW)

    bufs = (buf0, buf1)
    sems = (sem0, sem1)

    pltpu.sync_copy(gi_hbm.at[pl.ds(base, _CHP)], idx_v.at[jnp.int32(0)])
    pltpu.sync_copy(si_hbm.at[pl.ds(base, _CHP)], idx_v.at[jnp.int32(1)])
    pltpu.async_copy(u_hbm.at[idx_v.at[jnp.int32(0)]], buf0, sem0)

    def body(it, carry):
        for b in range(2):
            j = it * jnp.int32(2) + jnp.int32(b)
            jn = j + jnp.int32(1)
            nb = 1 - b
            gs = jnp.int32(2 * nb)
            ss = jnp.int32(2 * nb + 1)

            @pl.when(jn < jnp.int32(_NCHP))
            def _():
                offn = base + jn * jnp.int32(_CHP)
                pltpu.sync_copy(gi_hbm.at[pl.ds(offn, _CHP)], idx_v.at[gs])
                pltpu.sync_copy(si_hbm.at[pl.ds(offn, _CHP)], idx_v.at[ss])
                pltpu.async_copy(u_hbm.at[idx_v.at[gs]], bufs[nb], sems[nb])

            pltpu.make_async_copy(u_hbm.at[idx_v.at[jnp.int32(2 * b)]],
                                  bufs[b], sems[b]).wait()
            pltpu.sync_copy(bufs[b],
                            acc.at[idx_v.at[jnp.int32(2 * b + 1)]], add=True)
        return carry

    lax.fori_loop(jnp.int32(0), jnp.int32(_NCHP // 2), body, jnp.int32(0))
    last = jnp.int32(_NCHP - 1)
    lb = (_NCHP - 1) % 2
    pltpu.make_async_copy(u_hbm.at[idx_v.at[jnp.int32(2 * lb)]],
                          bufs[lb], sems[lb]).wait()
    pltpu.sync_copy(bufs[lb],
                    acc.at[idx_v.at[jnp.int32(2 * lb + 1)]], add=True)
    plsc.subcore_barrier()
    pltpu.sync_copy(acc.at[rows], out_hbm.at[cid, rows])


def _sc_prop(u, gi, si, zf):
    f = pl.kernel(
        _prop_body,
        out_type=jax.ShapeDtypeStruct((_NC, _NP, _C), jnp.float32),
        mesh=_mesh(),
        scratch_types=[
            pltpu.VMEM((4, _CHP), jnp.int32),
            pltpu.VMEM((_CHP, _C), jnp.float32),
            pltpu.VMEM((_CHP, _C), jnp.float32),
            pltpu.VMEM_SHARED((_NP, _C), jnp.float32),
            pltpu.SemaphoreType.DMA,
            pltpu.SemaphoreType.DMA,
        ],
    )
    return f(u, gi, si, zf)


def _prep_body(deg_ref, x_ref, s1_ref, s2_ref, u1_ref, u2_ref):
    dsum = deg_ref[0] + deg_ref[1]
    d1 = dsum[:, 0:1]
    d2 = dsum[:, 64:65]
    s1 = jnp.where(d1 > 0.5, lax.rsqrt(jnp.maximum(d1, 1.0)), 0.0)
    s2 = jnp.where(d2 > 0.5, lax.rsqrt(jnp.maximum(d2, 1.0)), 0.0)
    s1b = jnp.broadcast_to(s1, (_BR, _C))
    s2b = jnp.broadcast_to(s2, (_BR, _C))
    s1_ref[...] = s1b
    s2_ref[...] = s2b
    u1_ref[...] = s1b * x_ref[...]
    u2_ref[...] = s2b * x_ref[...]


def _prep(degp, x):
    fb = jax.ShapeDtypeStruct((_NP, _C), jnp.float32)
    return pl.pallas_call(
        _prep_body,
        grid=(_GRID,),
        in_specs=[
            pl.BlockSpec((_NC, _BR, _C), lambda i: (jnp.int32(0), i, jnp.int32(0))),
            pl.BlockSpec((_BR, _C), lambda i: (i, jnp.int32(0))),
        ],
        out_specs=[pl.BlockSpec((_BR, _C), lambda i: (i, jnp.int32(0)))] * 4,
        out_shape=[fb, fb, fb, fb],
    )(degp, x)


def _combine_body(alpha, beta, zp_ref, s_ref, tm1_ref, tm2_ref, t_ref, u_ref):
    z = zp_ref[0] + zp_ref[1]
    s = s_ref[...]
    p = (-2.0 / 3.0) * s * z - (1.0 / 3.0) * tm1_ref[...]
    t = alpha * p - beta * tm2_ref[...]
    t_ref[...] = t
    u_ref[...] = s * t


def _combine(alpha, beta, zp, s, tm1, tm2):
    fb = jax.ShapeDtypeStruct((_NP, _C), jnp.float32)
    return pl.pallas_call(
        functools.partial(_combine_body, alpha, beta),
        grid=(_GRID,),
        in_specs=[
            pl.BlockSpec((_NC, _BR, _C), lambda i: (jnp.int32(0), i, jnp.int32(0))),
            pl.BlockSpec((_BR, _C), lambda i: (i, jnp.int32(0))),
            pl.BlockSpec((_BR, _C), lambda i: (i, jnp.int32(0))),
            pl.BlockSpec((_BR, _C), lambda i: (i, jnp.int32(0))),
        ],
        out_specs=[pl.BlockSpec((_BR, _C), lambda i: (i, jnp.int32(0)))] * 2,
        out_shape=[fb, fb],
    )(zp, s, tm1, tm2)


def _mm_body(w1_ref, w2_ref, b1_ref, b2_ref, *refs):
    t_refs, out_ref = refs[:-1], refs[-1]
    h = _C // 2
    acc1 = jnp.zeros((_BR, h), jnp.float32) + b1_ref[...]
    acc2 = jnp.zeros((_BR, h), jnp.float32) + b2_ref[...]
    for k in range(_K):
        acc1 = acc1 + jnp.dot(t_refs[k][...], w1_ref[k],
                              preferred_element_type=jnp.float32)
        acc2 = acc2 + jnp.dot(t_refs[_K + k][...], w2_ref[k],
                              preferred_element_type=jnp.float32)
    out_ref[...] = jnp.concatenate([acc1, acc2], axis=-1)


def _matmul(W1, W2, b1, b2, T1, T2):
    h = _C // 2
    tspec = pl.BlockSpec((_BR, _C), lambda i: (i, jnp.int32(0)))
    return pl.pallas_call(
        _mm_body,
        grid=(_GRID,),
        in_specs=[
            pl.BlockSpec((_K, _C, h), lambda i: (jnp.int32(0), jnp.int32(0), jnp.int32(0))),
            pl.BlockSpec((_K, _C, h), lambda i: (jnp.int32(0), jnp.int32(0), jnp.int32(0))),
            pl.BlockSpec((1, h), lambda i: (jnp.int32(0), jnp.int32(0))),
            pl.BlockSpec((1, h), lambda i: (jnp.int32(0), jnp.int32(0))),
        ] + [tspec] * (2 * _K),
        out_specs=pl.BlockSpec((_BR, _C), lambda i: (i, jnp.int32(0))),
        out_shape=jax.ShapeDtypeStruct((_NP, _C), jnp.float32),
    )(W1, W2, b1, b2, *T1, *T2)


def kernel(x, edge_index, W1, b1, W2, b2):
    x = jnp.zeros((_NP, _C), jnp.float32).at[:_N].set(x.astype(jnp.float32))
    ei = edge_index.astype(jnp.int32)
    r, c = ei[0], ei[1]
    zf = jnp.zeros((_NP, _C), jnp.float32)
    masks = jnp.zeros((2, _CHD, _C), jnp.float32)
    masks = masks.at[0, :, : _C // 2].set(1.0).at[1, :, _C // 2 :].set(1.0)

    degp = _sc_degrees(r, c, zf, masks)
    s1, s2, u1, u2 = _prep(degp, x)

    stacks = []
    for s, u0, gi, si in ((s1, u1, c, r), (s2, u2, r, c)):
        T = [x]
        u = u0
        for k in range(1, _K):
            zp = _sc_prop(u, gi, si, zf)
            alpha, beta = (1.0, 0.0) if k == 1 else (2.0, 1.0)
            tm2 = T[k - 2] if k >= 2 else x
            t, u = _combine(alpha, beta, zp, s, T[k - 1], tm2)
            T.append(t)
        stacks.append(T)

    out = _matmul(W1.astype(jnp.float32), W2.astype(jnp.float32),
                  b1.reshape(1, -1).astype(jnp.float32),
                  b2.reshape(1, -1).astype(jnp.float32),
                  stacks[0], stacks[1])
    return out[:_N].astype(jnp.float64)


# speedup vs baseline: 2.3938x; 1.0024x over previous
import functools

import jax
import jax.numpy as jnp
from jax import lax
from jax.experimental import pallas as pl
from jax.experimental.pallas import tpu as pltpu
from jax.experimental.pallas import tpu_sc as plsc

_N = 10000
_E = 320000
_C = 128
_K = 5
_NC, _NS = 2, 16
_NP = 10240
_NW = _NC * _NS
_EPW = _E // _NW
_CH128 = 128
_EP = _NW * 10240
_CPP = 10240 // _CH128
_RPT = _NP // _NS
_BR = 1280
_GRID = _NP // _BR


def _mesh():
    return plsc.VectorSubcoreMesh(core_axis_name="c", subcore_axis_name="s",
                                  num_cores=_NC, num_subcores=_NS)


_CHD = 80
_NCHD = _EPW // _CHD


def _deg_body(r_hbm, c_hbm, zf_hbm, mask_hbm, out_hbm,
              idx_v, m1_v, m2_v, acc):
    cid = lax.axis_index("c")
    sid = lax.axis_index("s")
    wid = cid * jnp.int32(_NS) + sid
    rows = pl.ds(sid * jnp.int32(_RPT), _RPT)
    pltpu.sync_copy(zf_hbm.at[rows], acc.at[rows])
    pltpu.sync_copy(mask_hbm.at[jnp.int32(0)], m1_v)
    pltpu.sync_copy(mask_hbm.at[jnp.int32(1)], m2_v)
    plsc.subcore_barrier()
    base = wid * jnp.int32(_EPW)

    def body(i, carry):
        off = base + i * jnp.int32(_CHD)
        pltpu.sync_copy(r_hbm.at[pl.ds(off, _CHD)], idx_v.at[jnp.int32(0)])
        pltpu.sync_copy(c_hbm.at[pl.ds(off, _CHD)], idx_v.at[jnp.int32(1)])
        pltpu.sync_copy(m1_v, acc.at[idx_v.at[jnp.int32(0)]], add=True)
        pltpu.sync_copy(m2_v, acc.at[idx_v.at[jnp.int32(1)]], add=True)
        return carry

    lax.fori_loop(jnp.int32(0), jnp.int32(_NCHD), body, jnp.int32(0))
    plsc.subcore_barrier()
    pltpu.sync_copy(acc.at[rows], out_hbm.at[cid, rows])


def _sc_degrees(r, c, zf, masks):
    f = pl.kernel(
        _deg_body,
        out_type=jax.ShapeDtypeStruct((_NC, _NP, _C), jnp.float32),
        mesh=_mesh(),
        scratch_types=[
            pltpu.VMEM((2, _CHD), jnp.int32),
            pltpu.VMEM((_CHD, _C), jnp.float32),
            pltpu.VMEM((_CHD, _C), jnp.float32),
            pltpu.VMEM_SHARED((_NP, _C), jnp.float32),
        ],
    )
    return f(r, c, zf, masks)


_CHP = 80
_NCHP = _EPW // _CHP


def _prop_body(u_hbm, gi_hbm, si_hbm, zf_hbm, out_hbm,
               idx_v, buf0, buf1, buf2, acc, sem0, sem1, sem2):
    cid = lax.axis_index("c")
    sid = lax.axis_index("s")
    wid = cid * jnp.int32(_NS) + sid
    rows = pl.ds(sid * jnp.int32(_RPT), _RPT)
    pltpu.sync_copy(zf_hbm.at[rows], acc.at[rows])
    plsc.subcore_barrier()
    base = wid * jnp.int32(_EPW)

    bufs = (buf0, buf1, buf2)
    sems = (sem0, sem1, sem2)

    def load_idx(j, p):
        off = base + j * jnp.int32(_CHP)
        pltpu.sync_copy(gi_hbm.at[pl.ds(off, _CHP)], idx_v.at[jnp.int32(2 * p)])
        pltpu.sync_copy(si_hbm.at[pl.ds(off, _CHP)],
                        idx_v.at[jnp.int32(2 * p + 1)])

    def issue(p):
        pltpu.async_copy(u_hbm.at[idx_v.at[jnp.int32(2 * p)]], bufs[p], sems[p])

    def drain(p):
        pltpu.make_async_copy(u_hbm.at[idx_v.at[jnp.int32(2 * p)]],
                              bufs[p], sems[p]).wait()
        pltpu.sync_copy(bufs[p],
                        acc.at[idx_v.at[jnp.int32(2 * p + 1)]], add=True)

    for p in range(2):
        load_idx(jnp.int32(p), p)
        issue(p)

    def body(it, carry):
        for b in range(3):
            j = it * jnp.int32(3) + jnp.int32(b)
            jn = j + jnp.int32(2)
            nb = (b + 2) % 3

            @pl.when(jn < jnp.int32(_NCHP))
            def _():
                load_idx(jn, nb)
                issue(nb)

            drain(b)
        return carry

    lax.fori_loop(jnp.int32(0), jnp.int32(_NCHP // 3), body, jnp.int32(0))
    for j in range(_NCHP - _NCHP % 3, _NCHP):
        drain(j % 3)
    plsc.subcore_barrier()
    pltpu.sync_copy(acc.at[rows], out_hbm.at[cid, rows])


def _sc_prop(u, gi, si, zf):
    f = pl.kernel(
        _prop_body,
        out_type=jax.ShapeDtypeStruct((_NC, _NP, _C), jnp.float32),
        mesh=_mesh(),
        scratch_types=[
            pltpu.VMEM((6, _CHP), jnp.int32),
            pltpu.VMEM((_CHP, _C), jnp.float32),
            pltpu.VMEM((_CHP, _C), jnp.float32),
            pltpu.VMEM((_CHP, _C), jnp.float32),
            pltpu.VMEM_SHARED((_NP, _C), jnp.float32),
            pltpu.SemaphoreType.DMA,
            pltpu.SemaphoreType.DMA,
            pltpu.SemaphoreType.DMA,
        ],
    )
    return f(u, gi, si, zf)


def _prep_body(deg_ref, x_ref, s1_ref, s2_ref, u1_ref, u2_ref):
    dsum = deg_ref[0] + deg_ref[1]
    d1 = dsum[:, 0:1]
    d2 = dsum[:, 64:65]
    s1 = jnp.where(d1 > 0.5, lax.rsqrt(jnp.maximum(d1, 1.0)), 0.0)
    s2 = jnp.where(d2 > 0.5, lax.rsqrt(jnp.maximum(d2, 1.0)), 0.0)
    s1b = jnp.broadcast_to(s1, (_BR, _C))
    s2b = jnp.broadcast_to(s2, (_BR, _C))
    s1_ref[...] = s1b
    s2_ref[...] = s2b
    u1_ref[...] = s1b * x_ref[...]
    u2_ref[...] = s2b * x_ref[...]


def _prep(degp, x):
    fb = jax.ShapeDtypeStruct((_NP, _C), jnp.float32)
    return pl.pallas_call(
        _prep_body,
        grid=(_GRID,),
        in_specs=[
            pl.BlockSpec((_NC, _BR, _C), lambda i: (jnp.int32(0), i, jnp.int32(0))),
            pl.BlockSpec((_BR, _C), lambda i: (i, jnp.int32(0))),
        ],
        out_specs=[pl.BlockSpec((_BR, _C), lambda i: (i, jnp.int32(0)))] * 4,
        out_shape=[fb, fb, fb, fb],
    )(degp, x)


def _combine_body(alpha, beta, zp_ref, s_ref, tm1_ref, tm2_ref, t_ref, u_ref):
    z = zp_ref[0] + zp_ref[1]
    s = s_ref[...]
    p = (-2.0 / 3.0) * s * z - (1.0 / 3.0) * tm1_ref[...]
    t = alpha * p - beta * tm2_ref[...]
    t_ref[...] = t
    u_ref[...] = s * t


def _combine(alpha, beta, zp, s, tm1, tm2):
    fb = jax.ShapeDtypeStruct((_NP, _C), jnp.float32)
    return pl.pallas_call(
        functools.partial(_combine_body, alpha, beta),
        grid=(_GRID,),
        in_specs=[
            pl.BlockSpec((_NC, _BR, _C), lambda i: (jnp.int32(0), i, jnp.int32(0))),
            pl.BlockSpec((_BR, _C), lambda i: (i, jnp.int32(0))),
            pl.BlockSpec((_BR, _C), lambda i: (i, jnp.int32(0))),
            pl.BlockSpec((_BR, _C), lambda i: (i, jnp.int32(0))),
        ],
        out_specs=[pl.BlockSpec((_BR, _C), lambda i: (i, jnp.int32(0)))] * 2,
        out_shape=[fb, fb],
    )(zp, s, tm1, tm2)


def _mm_body(w1_ref, w2_ref, b1_ref, b2_ref, *refs):
    t_refs, out_ref = refs[:-1], refs[-1]
    h = _C // 2
    acc1 = jnp.zeros((_BR, h), jnp.float32) + b1_ref[...]
    acc2 = jnp.zeros((_BR, h), jnp.float32) + b2_ref[...]
    for k in range(_K):
        acc1 = acc1 + jnp.dot(t_refs[k][...], w1_ref[k],
                              preferred_element_type=jnp.float32)
        acc2 = acc2 + jnp.dot(t_refs[_K + k][...], w2_ref[k],
                              preferred_element_type=jnp.float32)
    out_ref[...] = jnp.concatenate([acc1, acc2], axis=-1)


def _matmul(W1, W2, b1, b2, T1, T2):
    h = _C // 2
    tspec = pl.BlockSpec((_BR, _C), lambda i: (i, jnp.int32(0)))
    return pl.pallas_call(
        _mm_body,
        grid=(_GRID,),
        in_specs=[
            pl.BlockSpec((_K, _C, h), lambda i: (jnp.int32(0), jnp.int32(0), jnp.int32(0))),
            pl.BlockSpec((_K, _C, h), lambda i: (jnp.int32(0), jnp.int32(0), jnp.int32(0))),
            pl.BlockSpec((1, h), lambda i: (jnp.int32(0), jnp.int32(0))),
            pl.BlockSpec((1, h), lambda i: (jnp.int32(0), jnp.int32(0))),
        ] + [tspec] * (2 * _K),
        out_specs=pl.BlockSpec((_BR, _C), lambda i: (i, jnp.int32(0))),
        out_shape=jax.ShapeDtypeStruct((_NP, _C), jnp.float32),
    )(W1, W2, b1, b2, *T1, *T2)


def kernel(x, edge_index, W1, b1, W2, b2):
    x = jnp.zeros((_NP, _C), jnp.float32).at[:_N].set(x.astype(jnp.float32))
    ei = edge_index.astype(jnp.int32)
    r, c = ei[0], ei[1]
    zf = jnp.zeros((_NP, _C), jnp.float32)

    masks = jnp.zeros((2, _CHD, _C), jnp.float32)
    masks = masks.at[0, :, : _C // 2].set(1.0).at[1, :, _C // 2 :].set(1.0)
    degp = _sc_degrees(r, c, zf, masks)
    s1, s2, u1, u2 = _prep(degp, x)

    stacks = []
    for s, u0, gi, si in ((s1, u1, c, r), (s2, u2, r, c)):
        T = [x]
        u = u0
        for k in range(1, _K):
            zp = _sc_prop(u, gi, si, zf)
            alpha, beta = (1.0, 0.0) if k == 1 else (2.0, 1.0)
            tm2 = T[k - 2] if k >= 2 else x
            t, u = _combine(alpha, beta, zp, s, T[k - 1], tm2)
            T.append(t)
        stacks.append(T)

    out = _matmul(W1.astype(jnp.float32), W2.astype(jnp.float32),
                  b1.reshape(1, -1).astype(jnp.float32),
                  b2.reshape(1, -1).astype(jnp.float32),
                  stacks[0], stacks[1])
    return out[:_N].astype(jnp.float64)
